# Initial kernel scaffold; baseline (speedup 1.0000x reference)
import jax
import jax.numpy as jnp
from jax.experimental import pallas as pl


def kernel(input_ids, attention_mask, embedding, W, b):
    raise NotImplementedError("write your pallas kernel here")


# speedup vs baseline: 2.0436x; 2.0436x over previous
import functools

import jax
import jax.numpy as jnp
from jax import lax
from jax.experimental import pallas as pl
from jax.experimental.pallas import tpu as pltpu
from jax.experimental.pallas import tpu_sc as plsc

B, L = 4096, 200
HIDDEN = 32
NUM_CLASSES = 2
HALF_L = L // 2

NUM_CORES, NUM_SUBCORES, LANES = 2, 16, 16
NUM_WORKERS = NUM_CORES * NUM_SUBCORES
SEQ_PER_W = B // NUM_WORKERS
OUT_PAD = LANES

_mesh = plsc.VectorSubcoreMesh(
    core_axis_name="c", subcore_axis_name="s",
    num_cores=NUM_CORES, num_subcores=NUM_SUBCORES,
)


@functools.partial(
    pl.kernel,
    out_type=jax.ShapeDtypeStruct((B, OUT_PAD), jnp.float32),
    mesh=_mesh,
    compiler_params=pltpu.CompilerParams(
        needs_layout_passes=False, use_tc_tiling_on_sc=False),
    scratch_types=[
        pltpu.VMEM((SEQ_PER_W, 2, HALF_L), jnp.int32),
        pltpu.VMEM((SEQ_PER_W, L), jnp.float32),
        pltpu.VMEM((L, HIDDEN), jnp.float32),
        pltpu.VMEM((SEQ_PER_W, OUT_PAD), jnp.float32),
        pltpu.VMEM((NUM_CLASSES, HIDDEN), jnp.float32),
        pltpu.VMEM((LANES,), jnp.float32),
        pltpu.SemaphoreType.DMA,
    ],
)
def _sc_classify(ids_hbm, mask_hbm, emb_hbm, w_hbm, b_hbm, out_hbm,
                 ids_v, mask_v, rows_v, out_v, w_v, b_v, sem):
    wid = lax.axis_index("s") * NUM_CORES + lax.axis_index("c")
    base = wid * SEQ_PER_W

    pltpu.sync_copy(ids_hbm.at[pl.ds(base, SEQ_PER_W)], ids_v)
    pltpu.sync_copy(mask_hbm.at[pl.ds(base, SEQ_PER_W)], mask_v)
    pltpu.sync_copy(w_hbm, w_v)
    pltpu.sync_copy(b_hbm, b_v)

    w00 = w_v[0, pl.ds(0, LANES)]
    w01 = w_v[0, pl.ds(LANES, LANES)]
    w10 = w_v[1, pl.ds(0, LANES)]
    w11 = w_v[1, pl.ds(LANES, LANES)]
    b_vec = b_v[pl.ds(0, LANES)]
    b0 = b_vec[0]
    b1 = b_vec[1]
    zero = jnp.zeros((LANES,), jnp.float32)
    lane = lax.iota(jnp.int32, LANES)

    def seq_body(j, carry):
        cp0 = pltpu.async_copy(
            emb_hbm.at[ids_v.at[j, 0]], rows_v.at[pl.ds(0, HALF_L)], sem)
        cp1 = pltpu.async_copy(
            emb_hbm.at[ids_v.at[j, 1]], rows_v.at[pl.ds(HALF_L, HALF_L)], sem)
        cp0.wait()
        cp1.wait()

        acc0 = zero
        acc1 = zero
        msum = jnp.float32(0.0)
        for g in range(L // LANES):
            mvec = mask_v[j, pl.ds(g * LANES, LANES)]
            for i in range(LANES):
                t = g * LANES + i
                m = mvec[i]
                acc0 = acc0 + rows_v[t, pl.ds(0, LANES)] * m
                acc1 = acc1 + rows_v[t, pl.ds(LANES, LANES)] * m
                msum = msum + m
        rem = L % LANES
        if rem:
            mvec = mask_v[j, pl.ds(L - LANES, LANES)]
            for i in range(LANES - rem, LANES):
                t = L - LANES + i
                m = mvec[i]
                acc0 = acc0 + rows_v[t, pl.ds(0, LANES)] * m
                acc1 = acc1 + rows_v[t, pl.ds(LANES, LANES)] * m
                msum = msum + m

        inv = jnp.full((LANES,), 1.0, jnp.float32) / jnp.broadcast_to(msum, (LANES,))
        p0 = acc0 * inv
        p1 = acc1 * inv
        l0 = jnp.sum(p0 * w00) + jnp.sum(p1 * w01) + b0
        l1 = jnp.sum(p0 * w10) + jnp.sum(p1 * w11) + b1
        out_row = jnp.where(lane == 0, l0, jnp.where(lane == 1, l1, 0.0))
        out_v[j, pl.ds(0, LANES)] = out_row
        return carry

    lax.fori_loop(0, SEQ_PER_W, seq_body, jnp.int32(0))

    pltpu.sync_copy(out_v, out_hbm.at[pl.ds(base, SEQ_PER_W)])


def kernel(input_ids, attention_mask, embedding, W, b):
    ids = input_ids.astype(jnp.int32).reshape(B, 2, HALF_L)
    b_pad = jnp.zeros((LANES,), jnp.float32).at[:NUM_CLASSES].set(
        b.astype(jnp.float32))
    padded = _sc_classify(ids, attention_mask.astype(jnp.float32),
                          embedding, W.astype(jnp.float32), b_pad)
    return padded[:, :NUM_CLASSES]


# speedup vs baseline: 2.4234x; 1.1859x over previous
import functools

import jax
import jax.numpy as jnp
from jax import lax
from jax.experimental import pallas as pl
from jax.experimental.pallas import tpu as pltpu
from jax.experimental.pallas import tpu_sc as plsc

B, L = 4096, 200
HIDDEN = 32
NUM_CLASSES = 2
HALF_L = L // 2

NUM_CORES, NUM_SUBCORES, LANES = 2, 16, 16
NUM_WORKERS = NUM_CORES * NUM_SUBCORES
SEQ_PER_W = B // NUM_WORKERS
OUT_PAD = LANES
NBUF = 4
FULL_GROUPS = L // LANES
REM = L % LANES

_mesh = plsc.VectorSubcoreMesh(
    core_axis_name="c", subcore_axis_name="s",
    num_cores=NUM_CORES, num_subcores=NUM_SUBCORES,
)


@functools.partial(
    pl.kernel,
    out_type=jax.ShapeDtypeStruct((B, OUT_PAD), jnp.float32),
    mesh=_mesh,
    compiler_params=pltpu.CompilerParams(
        needs_layout_passes=False, use_tc_tiling_on_sc=False),
    scratch_types=[
        pltpu.VMEM((SEQ_PER_W, 2, HALF_L), jnp.int32),
        pltpu.VMEM((SEQ_PER_W, L), jnp.float32),
        pltpu.VMEM((NBUF, L, HIDDEN), jnp.float32),
        pltpu.VMEM((SEQ_PER_W, OUT_PAD), jnp.float32),
        pltpu.VMEM((NUM_CLASSES, HIDDEN), jnp.float32),
        pltpu.VMEM((LANES,), jnp.float32),
        pltpu.SemaphoreType.DMA,
        pltpu.SemaphoreType.DMA,
        pltpu.SemaphoreType.DMA,
        pltpu.SemaphoreType.DMA,
    ],
)
def _sc_classify(ids_hbm, mask_hbm, emb_hbm, w_hbm, b_hbm, out_hbm,
                 ids_v, mask_v, rows_v, out_v, w_v, b_v,
                 sem0, sem1, sem2, sem3):
    sems = (sem0, sem1, sem2, sem3)
    wid = lax.axis_index("s") * NUM_CORES + lax.axis_index("c")
    base = wid * SEQ_PER_W

    pltpu.sync_copy(ids_hbm.at[pl.ds(base, SEQ_PER_W)], ids_v)
    pltpu.sync_copy(mask_hbm.at[pl.ds(base, SEQ_PER_W)], mask_v)
    pltpu.sync_copy(w_hbm, w_v)
    pltpu.sync_copy(b_hbm, b_v)

    w00 = w_v[0, pl.ds(0, LANES)]
    w01 = w_v[0, pl.ds(LANES, LANES)]
    w10 = w_v[1, pl.ds(0, LANES)]
    w11 = w_v[1, pl.ds(LANES, LANES)]
    b_vec = b_v[pl.ds(0, LANES)]
    b0 = b_vec[0]
    b1 = b_vec[1]
    zero = jnp.zeros((LANES,), jnp.float32)
    lane = lax.iota(jnp.int32, LANES)

    def copies(j, buf):
        return (
            pltpu.make_async_copy(
                emb_hbm.at[ids_v.at[j, 0]],
                rows_v.at[buf, pl.ds(0, HALF_L)], sems[buf]),
            pltpu.make_async_copy(
                emb_hbm.at[ids_v.at[j, 1]],
                rows_v.at[buf, pl.ds(HALF_L, HALF_L)], sems[buf]),
        )

    def fire(j, buf):
        for cp in copies(j, buf):
            cp.start()

    def drain(j, buf):
        for cp in copies(j, buf):
            cp.wait()

    def compute(j, buf):
        def group_body(g, carry):
            acc0, acc1, msvec = carry
            mvec = mask_v[j, pl.ds(g * LANES, LANES)]
            t0 = g * LANES
            for i in range(LANES):
                m = mvec[i]
                acc0 = acc0 + rows_v[buf, t0 + i, pl.ds(0, LANES)] * m
                acc1 = acc1 + rows_v[buf, t0 + i, pl.ds(LANES, LANES)] * m
            return (acc0, acc1, msvec + mvec)

        acc0, acc1, msvec = lax.fori_loop(
            0, FULL_GROUPS, group_body, (zero, zero, zero))

        mvec = mask_v[j, pl.ds(L - LANES, LANES)]
        for i in range(LANES - REM, LANES):
            t = L - LANES + i
            m = mvec[i]
            acc0 = acc0 + rows_v[buf, t, pl.ds(0, LANES)] * m
            acc1 = acc1 + rows_v[buf, t, pl.ds(LANES, LANES)] * m
        msvec = msvec + jnp.where(lane >= LANES - REM, mvec, 0.0)

        msum = jnp.sum(msvec)
        inv = jnp.full((LANES,), 1.0, jnp.float32) / jnp.broadcast_to(
            msum, (LANES,))
        p0 = acc0 * inv
        p1 = acc1 * inv
        l0 = jnp.sum(p0 * w00) + jnp.sum(p1 * w01) + b0
        l1 = jnp.sum(p0 * w10) + jnp.sum(p1 * w11) + b1
        out_v[j, pl.ds(0, LANES)] = jnp.where(
            lane == 0, l0, jnp.where(lane == 1, l1, 0.0))

    for buf in range(NBUF):
        fire(jnp.int32(buf), buf)

    def ring_body(g, carry):
        j0 = g * NBUF
        for buf in range(NBUF):
            j = j0 + buf
            drain(j, buf)
            compute(j, buf)
            nxt = j + NBUF

            @pl.when(nxt < SEQ_PER_W)
            def _():
                fire(nxt, buf)
        return carry

    lax.fori_loop(0, SEQ_PER_W // NBUF, ring_body, jnp.int32(0))

    pltpu.sync_copy(out_v, out_hbm.at[pl.ds(base, SEQ_PER_W)])


def kernel(input_ids, attention_mask, embedding, W, b):
    ids = input_ids.astype(jnp.int32).reshape(B, 2, HALF_L)
    b_pad = jnp.zeros((LANES,), jnp.float32).at[:NUM_CLASSES].set(
        b.astype(jnp.float32))
    padded = _sc_classify(ids, attention_mask.astype(jnp.float32),
                          embedding, W.astype(jnp.float32), b_pad)
    return padded[:, :NUM_CLASSES]
